# baseline (speedup 1.0000x reference)
import functools

import jax
import jax.numpy as jnp
from jax import lax
from jax.experimental import pallas as pl
from jax.experimental.pallas import tpu as pltpu
from jax.experimental.pallas import tpu_sc as plsc

N_TOKENS = 8192
FEAT = 4096
LANES = 16

NUM_CORES = 2
NUM_SUBCORES = 16
NUM_WORKERS = NUM_CORES * NUM_SUBCORES
ROWS_PER_WORKER = N_TOKENS // NUM_WORKERS
ROWS_PER_BLOCK = 8
NUM_BLOCKS = ROWS_PER_WORKER // ROWS_PER_BLOCK


BLOCK_ELEMS = ROWS_PER_BLOCK * FEAT


def _permute_body(x_hbm, perm_hbm, out_hbm, perm_v, in_v, out_v):
    wid = lax.axis_index("s") * NUM_CORES + lax.axis_index("c")
    base = wid * ROWS_PER_WORKER

    pltpu.sync_copy(perm_hbm, perm_v)

    row_offs = [jnp.full((LANES,), r * FEAT, jnp.int32)
                for r in range(ROWS_PER_BLOCK)]

    def block_body(g, carry):
        elem0 = (base + g * ROWS_PER_BLOCK) * FEAT
        pltpu.sync_copy(x_hbm.at[pl.ds(elem0, BLOCK_ELEMS)], in_v)

        def jbody(j, carry):
            idx = perm_v[pl.ds(j * LANES, LANES)]
            for r in range(ROWS_PER_BLOCK):
                out_v[pl.ds(r * FEAT + j * LANES, LANES)] = plsc.load_gather(
                    in_v, [idx + row_offs[r]])
            return carry

        lax.fori_loop(0, FEAT // LANES, jbody, 0)
        pltpu.sync_copy(out_v, out_hbm.at[pl.ds(elem0, BLOCK_ELEMS)])
        return carry

    lax.fori_loop(0, NUM_BLOCKS, block_body, 0)


@jax.jit
def kernel(x, permute):
    perm = permute.astype(jnp.int32)
    mesh = plsc.VectorSubcoreMesh(
        core_axis_name="c", subcore_axis_name="s",
        num_cores=NUM_CORES, num_subcores=NUM_SUBCORES)
    run = pl.kernel(
        _permute_body,
        out_type=jax.ShapeDtypeStruct((N_TOKENS * FEAT,), jnp.float32),
        mesh=mesh,
        compiler_params=pltpu.CompilerParams(needs_layout_passes=False),
        scratch_types=[
            pltpu.VMEM((FEAT,), jnp.int32),
            pltpu.VMEM((BLOCK_ELEMS,), jnp.float32),
            pltpu.VMEM((BLOCK_ELEMS,), jnp.float32),
        ],
    )
    return run(x.reshape(-1), perm).reshape(N_TOKENS, FEAT)


# speedup vs baseline: 1.9741x; 1.9741x over previous
import jax
import jax.numpy as jnp
from jax import lax
from jax.experimental import pallas as pl
from jax.experimental.pallas import tpu as pltpu
from jax.experimental.pallas import tpu_sc as plsc

N_TOKENS = 8192
FEAT = 4096
LANES = 16

NUM_CORES = 2
NUM_SUBCORES = 16
NUM_WORKERS = NUM_CORES * NUM_SUBCORES
ROWS_PER_WORKER = N_TOKENS // NUM_WORKERS
ROWS_PER_BLOCK = 4
NUM_BLOCKS = ROWS_PER_WORKER // ROWS_PER_BLOCK
BLOCK_ELEMS = ROWS_PER_BLOCK * FEAT
NBUF = 2


def _permute_body(x_hbm, perm_hbm, out_hbm, perm_v,
                  in0, in1, out0, out1, sin0, sin1, sout0, sout1):
    wid = lax.axis_index("s") * NUM_CORES + lax.axis_index("c")
    base = wid * ROWS_PER_WORKER

    pltpu.sync_copy(perm_hbm, perm_v)

    ins, outs = [in0, in1], [out0, out1]
    sins, souts = [sin0, sin1], [sout0, sout1]

    def x_slice(g):
        return x_hbm.at[pl.ds((base + g * ROWS_PER_BLOCK) * FEAT, BLOCK_ELEMS)]

    def y_slice(g):
        return out_hbm.at[pl.ds((base + g * ROWS_PER_BLOCK) * FEAT, BLOCK_ELEMS)]

    row_offs = [jnp.full((LANES,), r * FEAT, jnp.int32)
                for r in range(ROWS_PER_BLOCK)]

    def gather_block(in_v, out_v):
        @plsc.parallel_loop(0, FEAT // LANES, 1, unroll=4)
        def jbody(j):
            idx = perm_v[pl.ds(j * LANES, LANES)]
            for r in range(ROWS_PER_BLOCK):
                out_v[pl.ds(r * FEAT + j * LANES, LANES)] = plsc.load_gather(
                    in_v, [idx + row_offs[r]])

    for b in range(NBUF):
        pltpu.async_copy(x_slice(b), ins[b], sins[b])

    def outer(t, carry):
        for b in range(NBUF):
            g = t * NBUF + b
            pltpu.make_async_copy(x_slice(g), ins[b], sins[b]).wait()

            @pl.when(t > 0)
            def _wait_out():
                pltpu.make_async_copy(outs[b], y_slice(g), souts[b]).wait()

            gather_block(ins[b], outs[b])
            pltpu.async_copy(outs[b], y_slice(g), souts[b])

            @pl.when(g + NBUF < NUM_BLOCKS)
            def _next_in():
                pltpu.async_copy(x_slice(g + NBUF), ins[b], sins[b])
        return carry

    lax.fori_loop(0, NUM_BLOCKS // NBUF, outer, 0)

    for b in range(NBUF):
        g = NUM_BLOCKS - NBUF + b
        pltpu.make_async_copy(outs[b], y_slice(g), souts[b]).wait()


@jax.jit
def kernel(x, permute):
    perm = permute.astype(jnp.int32)
    mesh = plsc.VectorSubcoreMesh(
        core_axis_name="c", subcore_axis_name="s",
        num_cores=NUM_CORES, num_subcores=NUM_SUBCORES)
    run = pl.kernel(
        _permute_body,
        out_type=jax.ShapeDtypeStruct((N_TOKENS * FEAT,), jnp.float32),
        mesh=mesh,
        compiler_params=pltpu.CompilerParams(needs_layout_passes=False),
        scratch_types=[
            pltpu.VMEM((FEAT,), jnp.int32),
            pltpu.VMEM((BLOCK_ELEMS,), jnp.float32),
            pltpu.VMEM((BLOCK_ELEMS,), jnp.float32),
            pltpu.VMEM((BLOCK_ELEMS,), jnp.float32),
            pltpu.VMEM((BLOCK_ELEMS,), jnp.float32),
            pltpu.SemaphoreType.DMA,
            pltpu.SemaphoreType.DMA,
            pltpu.SemaphoreType.DMA,
            pltpu.SemaphoreType.DMA,
        ],
    )
    return run(x.reshape(-1), perm).reshape(N_TOKENS, FEAT)


# speedup vs baseline: 5.7550x; 2.9152x over previous
import jax
import jax.numpy as jnp
from jax import lax
from jax.experimental import pallas as pl
from jax.experimental.pallas import tpu as pltpu
from jax.experimental.pallas import tpu_sc as plsc

N_TOKENS = 8192
FEAT = 4096
LANES = 16

NUM_CORES = 2
NUM_SUBCORES = 16
NUM_WORKERS = NUM_CORES * NUM_SUBCORES
ROWS_PER_WORKER = N_TOKENS // NUM_WORKERS
ROWS_PER_BLOCK = 4
NUM_BLOCKS = ROWS_PER_WORKER // ROWS_PER_BLOCK
NBUF = 2


def _permute_body(x_hbm, perm_hbm, out_hbm, perm_v,
                  in0, in1, out0, out1, sin0, sin1, sout0, sout1):
    wid = lax.axis_index("s") * NUM_CORES + lax.axis_index("c")
    base = wid * ROWS_PER_WORKER

    pltpu.sync_copy(perm_hbm, perm_v)

    ins, outs = [in0, in1], [out0, out1]
    sins, souts = [sin0, sin1], [sout0, sout1]

    def x_slice(g):
        return x_hbm.at[pl.ds(base + g * ROWS_PER_BLOCK, ROWS_PER_BLOCK)]

    def y_slice(g):
        return out_hbm.at[pl.ds(base + g * ROWS_PER_BLOCK, ROWS_PER_BLOCK)]

    row_ids = [jnp.full((LANES,), r, jnp.int32) for r in range(ROWS_PER_BLOCK)]

    def gather_block(in_v, out_v):
        @plsc.parallel_loop(0, FEAT // LANES, 1, unroll=4)
        def jbody(j):
            idx = perm_v[pl.ds(j * LANES, LANES)]
            for r in range(ROWS_PER_BLOCK):
                out_v[r, pl.ds(j * LANES, LANES)] = plsc.load_gather(
                    in_v, [row_ids[r], idx])

    for b in range(NBUF):
        pltpu.async_copy(x_slice(b), ins[b], sins[b])

    def outer(t, carry):
        for b in range(NBUF):
            g = t * NBUF + b
            pltpu.make_async_copy(x_slice(g), ins[b], sins[b]).wait()

            @pl.when(t > 0)
            def _wait_out():
                pltpu.make_async_copy(outs[b], y_slice(g), souts[b]).wait()

            gather_block(ins[b], outs[b])
            pltpu.async_copy(outs[b], y_slice(g), souts[b])

            @pl.when(g + NBUF < NUM_BLOCKS)
            def _next_in():
                pltpu.async_copy(x_slice(g + NBUF), ins[b], sins[b])
        return carry

    lax.fori_loop(0, NUM_BLOCKS // NBUF, outer, 0)

    for b in range(NBUF):
        g = NUM_BLOCKS - NBUF + b
        pltpu.make_async_copy(outs[b], y_slice(g), souts[b]).wait()


@jax.jit
def kernel(x, permute):
    perm = permute.astype(jnp.int32)
    mesh = plsc.VectorSubcoreMesh(
        core_axis_name="c", subcore_axis_name="s",
        num_cores=NUM_CORES, num_subcores=NUM_SUBCORES)
    run = pl.kernel(
        _permute_body,
        out_type=jax.ShapeDtypeStruct((N_TOKENS, FEAT), jnp.float32),
        mesh=mesh,
        compiler_params=pltpu.CompilerParams(needs_layout_passes=False),
        scratch_types=[
            pltpu.VMEM((FEAT,), jnp.int32),
            pltpu.VMEM((ROWS_PER_BLOCK, FEAT), jnp.float32),
            pltpu.VMEM((ROWS_PER_BLOCK, FEAT), jnp.float32),
            pltpu.VMEM((ROWS_PER_BLOCK, FEAT), jnp.float32),
            pltpu.VMEM((ROWS_PER_BLOCK, FEAT), jnp.float32),
            pltpu.SemaphoreType.DMA,
            pltpu.SemaphoreType.DMA,
            pltpu.SemaphoreType.DMA,
            pltpu.SemaphoreType.DMA,
        ],
    )
    return run(x, perm)


# speedup vs baseline: 5.9754x; 1.0383x over previous
import jax
import jax.numpy as jnp
from jax import lax
from jax.experimental import pallas as pl
from jax.experimental.pallas import tpu as pltpu
from jax.experimental.pallas import tpu_sc as plsc

N_TOKENS = 8192
FEAT = 4096
LANES = 16

NUM_CORES = 2
NUM_SUBCORES = 16
NUM_WORKERS = NUM_CORES * NUM_SUBCORES
ROWS_PER_WORKER = N_TOKENS // NUM_WORKERS
ROWS_PER_BLOCK = 4
NUM_BLOCKS = ROWS_PER_WORKER // ROWS_PER_BLOCK
NBUF = 3


def _permute_body(x_hbm, perm_hbm, out_hbm, perm_v,
                  in0, in1, in2, out0, out1, out2,
                  sin0, sin1, sin2, sout0, sout1, sout2):
    wid = lax.axis_index("s") * NUM_CORES + lax.axis_index("c")
    base = wid * ROWS_PER_WORKER

    pltpu.sync_copy(perm_hbm, perm_v)

    ins, outs = [in0, in1, in2], [out0, out1, out2]
    sins, souts = [sin0, sin1, sin2], [sout0, sout1, sout2]

    def x_slice(g):
        return x_hbm.at[pl.ds(base + g * ROWS_PER_BLOCK, ROWS_PER_BLOCK)]

    def y_slice(g):
        return out_hbm.at[pl.ds(base + g * ROWS_PER_BLOCK, ROWS_PER_BLOCK)]

    row_ids = [jnp.full((LANES,), r, jnp.int32) for r in range(ROWS_PER_BLOCK)]

    def gather_block(in_v, out_v):
        @plsc.parallel_loop(0, FEAT // LANES, 1, unroll=4)
        def jbody(j):
            idx = perm_v[pl.ds(j * LANES, LANES)]
            for r in range(ROWS_PER_BLOCK):
                out_v[r, pl.ds(j * LANES, LANES)] = plsc.load_gather(
                    in_v, [row_ids[r], idx])

    for b in range(NBUF):
        pltpu.async_copy(x_slice(b), ins[b], sins[b])

    def outer(t, carry):
        for b in range(NBUF):
            g = t * NBUF + b

            @pl.when(g < NUM_BLOCKS)
            def _step():
                pltpu.make_async_copy(x_slice(g), ins[b], sins[b]).wait()

                @pl.when(t > 0)
                def _wait_out():
                    pltpu.make_async_copy(outs[b], y_slice(g), souts[b]).wait()

                gather_block(ins[b], outs[b])
                pltpu.async_copy(outs[b], y_slice(g), souts[b])

                @pl.when(g + NBUF < NUM_BLOCKS)
                def _next_in():
                    pltpu.async_copy(x_slice(g + NBUF), ins[b], sins[b])
        return carry

    n_outer = (NUM_BLOCKS + NBUF - 1) // NBUF
    lax.fori_loop(0, n_outer, outer, 0)

    for b in range(NBUF):
        g = ((NUM_BLOCKS - 1 - b) // NBUF) * NBUF + b
        pltpu.make_async_copy(outs[b], y_slice(g), souts[b]).wait()


@jax.jit
def kernel(x, permute):
    perm = permute.astype(jnp.int32)
    mesh = plsc.VectorSubcoreMesh(
        core_axis_name="c", subcore_axis_name="s",
        num_cores=NUM_CORES, num_subcores=NUM_SUBCORES)
    run = pl.kernel(
        _permute_body,
        out_type=jax.ShapeDtypeStruct((N_TOKENS, FEAT), jnp.float32),
        mesh=mesh,
        compiler_params=pltpu.CompilerParams(needs_layout_passes=False),
        scratch_types=[
            pltpu.VMEM((FEAT,), jnp.int32),
            pltpu.VMEM((ROWS_PER_BLOCK, FEAT), jnp.float32),
            pltpu.VMEM((ROWS_PER_BLOCK, FEAT), jnp.float32),
            pltpu.VMEM((ROWS_PER_BLOCK, FEAT), jnp.float32),
            pltpu.VMEM((ROWS_PER_BLOCK, FEAT), jnp.float32),
            pltpu.VMEM((ROWS_PER_BLOCK, FEAT), jnp.float32),
            pltpu.VMEM((ROWS_PER_BLOCK, FEAT), jnp.float32),
            pltpu.SemaphoreType.DMA,
            pltpu.SemaphoreType.DMA,
            pltpu.SemaphoreType.DMA,
            pltpu.SemaphoreType.DMA,
            pltpu.SemaphoreType.DMA,
            pltpu.SemaphoreType.DMA,
        ],
    )
    return run(x, perm)
